# baseline (speedup 1.0000x reference)
import functools

import jax
import jax.numpy as jnp
from jax.experimental import pallas as pl
from jax.experimental.pallas import tpu as pltpu
from jax.experimental.pallas import tpu_sc as plsc

DEPTH = 3
N_MOLS = 100

_NC = 2
_NS = 16
_NW = _NC * _NS

_BF = jnp.bfloat16


def _pick_ring(per_w):
    for g in range(128, 0, -8):
        if per_w % g == 0:
            for nb in (5, 4, 3, 2, 1):
                if per_w % (g * nb) == 0:
                    return g, nb
    raise ValueError(per_w)


def _rt16(x):
    u = jax.lax.bitcast_convert_type(x, jnp.uint32)
    u = u + jnp.uint32(0x7FFF) + ((u >> 16) & jnp.uint32(1))
    return u >> 16


def _pack2(lo, hi):
    return jax.lax.bitcast_convert_type(
        _rt16(lo) | (_rt16(hi) << 16), jnp.int32
    )


def _unpack(p):
    u = jax.lax.bitcast_convert_type(p, jnp.uint32)
    lo = jax.lax.bitcast_convert_type(u << 16, jnp.float32)
    hi = jax.lax.bitcast_convert_type(
        u & jnp.uint32(0xFFFF0000), jnp.float32
    )
    return lo, hi


def _sc_gather_multi(jobs):
    n = jobs[0][1].shape[0]
    d = jobs[0][0].shape[1]
    per_w = n // _NW
    assert per_w * _NW == n and per_w % 8 == 0
    for t_, i_ in jobs:
        assert i_.shape[0] == n and t_.shape[1] == d
    gchunk, nbuf = _pick_ring(per_w)
    group = gchunk * nbuf
    n_groups = per_w // group
    mesh = plsc.VectorSubcoreMesh(core_axis_name="c", subcore_axis_name="s")
    njobs = len(jobs)

    @functools.partial(
        pl.kernel,
        mesh=mesh,
        out_type=[
            jax.ShapeDtypeStruct((n, d), jobs[0][0].dtype)
            for _ in range(njobs)
        ],
        scratch_types=[
            pltpu.VMEM((group,), jnp.int32),
            pltpu.VMEM((nbuf, gchunk, d), jobs[0][0].dtype),
            pltpu.SemaphoreType.DMA,
        ]
        + [pltpu.SemaphoreType.DMA] * nbuf,
    )
    def k(*refs):
        tables = refs[:njobs]
        idxs = refs[njobs:2 * njobs]
        outs = refs[2 * njobs:3 * njobs]
        idx_v, rows_v, gsem = refs[3 * njobs:3 * njobs + 3]
        wsems = refs[3 * njobs + 3:]
        wid = jax.lax.axis_index("s") * _NC + jax.lax.axis_index("c")
        base = wid * per_w

        def do_group(table_hbm, idx_hbm, out_hbm, g, wait_wb):
            gbase = pl.multiple_of(base + g * group, 8)
            if wait_wb:
                for b in range(nbuf):
                    off = pl.multiple_of(gbase + b * gchunk, 8)
                    pltpu.make_async_copy(
                        rows_v.at[b], out_hbm.at[pl.ds(off, gchunk)], wsems[b]
                    ).wait()
            pltpu.sync_copy(idx_hbm.at[pl.ds(gbase, group)], idx_v)
            handles = [
                pltpu.async_copy(
                    table_hbm.at[idx_v.at[pl.ds(b * gchunk, gchunk)]],
                    rows_v.at[b],
                    gsem,
                )
                for b in range(nbuf)
            ]
            for b, h in enumerate(handles):
                h.wait()
                off = pl.multiple_of(gbase + b * gchunk, 8)
                pltpu.async_copy(
                    rows_v.at[b], out_hbm.at[pl.ds(off, gchunk)], wsems[b]
                )

        for j in range(njobs):
            do_group(tables[j], idxs[j], outs[j], 0, wait_wb=(j > 0))

            @pl.loop(1, n_groups)
            def _(g, _j=j):
                do_group(tables[_j], idxs[_j], outs[_j], g, True)

        last = pl.multiple_of(base + (n_groups - 1) * group, 8)
        for b in range(nbuf):
            off = pl.multiple_of(last + b * gchunk, 8)
            pltpu.make_async_copy(
                rows_v.at[b], outs[-1].at[pl.ds(off, gchunk)], wsems[b]
            ).wait()

    outs = k(*[t for t, _ in jobs], *[i for _, i in jobs])
    if not isinstance(outs, (list, tuple)):
        outs = [outs]
    return list(outs)


def _sc_gather(table, idx):
    return _sc_gather_multi([(table, idx)])[0]


_BOND_BLK = 2000
_ATOM_BLK = 200


def _tc_init(f_bonds, w_i_t, w_h_t):
    n, fdim = f_bonds.shape
    h = w_i_t.shape[1]
    hh = h // 2

    def body(fb, wi, wh, inp_ref, msgh_ref):
        inp = jnp.dot(
            fb[...], wi[...], preferred_element_type=jnp.float32
        )
        inp_ref[...] = _pack2(inp[:, :hh], inp[:, hh:])
        msgh = jnp.dot(
            jnp.maximum(inp, 0.0).astype(_BF), wh[...],
            preferred_element_type=jnp.float32,
        )
        msgh_ref[...] = _pack2(msgh[:, :hh], msgh[:, hh:])

    return pl.pallas_call(
        body,
        grid=(n // _BOND_BLK,),
        in_specs=[
            pl.BlockSpec((_BOND_BLK, fdim), lambda i: (i, 0)),
            pl.BlockSpec((fdim, h), lambda i: (0, 0)),
            pl.BlockSpec((h, h), lambda i: (0, 0)),
        ],
        out_specs=[
            pl.BlockSpec((_BOND_BLK, hh), lambda i: (i, 0)),
            pl.BlockSpec((_BOND_BLK, hh), lambda i: (i, 0)),
        ],
        out_shape=[
            jax.ShapeDtypeStruct((n, hh), jnp.int32),
            jax.ShapeDtypeStruct((n, hh), jnp.int32),
        ],
    )(f_bonds, w_i_t, w_h_t)


def _tc_segsum(nei):
    a, k, hh = nei.shape

    def body(n_ref, o_ref):
        lo, hi = _unpack(n_ref[...])
        o_ref[...] = _pack2(jnp.sum(lo, axis=1), jnp.sum(hi, axis=1))

    return pl.pallas_call(
        body,
        grid=(a // _ATOM_BLK,),
        in_specs=[pl.BlockSpec((_ATOM_BLK, k, hh), lambda i: (i, 0, 0))],
        out_specs=pl.BlockSpec((_ATOM_BLK, hh), lambda i: (i, 0)),
        out_shape=jax.ShapeDtypeStruct((a, hh), jnp.int32),
    )(nei)


def _tc_combine(inp, g1, rev, w_h_t):
    n, hh = inp.shape
    matmul = w_h_t is not None

    def body(*refs):
        if matmul:
            inp_ref, g1_ref, rev_ref, wh_ref, o_ref = refs
        else:
            inp_ref, g1_ref, rev_ref, o_ref = refs
        i_lo, i_hi = _unpack(inp_ref[...])
        g_lo, g_hi = _unpack(g1_ref[...])
        r_lo, r_hi = _unpack(rev_ref[...])
        m_lo = jnp.maximum(i_lo + g_lo - r_lo, 0.0)
        m_hi = jnp.maximum(i_hi + g_hi - r_hi, 0.0)
        if matmul:
            wh = wh_ref[...]
            out = jnp.dot(
                m_lo.astype(_BF), wh[:hh], preferred_element_type=jnp.float32
            ) + jnp.dot(
                m_hi.astype(_BF), wh[hh:], preferred_element_type=jnp.float32
            )
            o_ref[...] = _pack2(out[:, :hh], out[:, hh:])
        else:
            o_ref[...] = _pack2(m_lo, m_hi)

    row_spec = pl.BlockSpec((_BOND_BLK, hh), lambda i: (i, 0))
    in_specs = [row_spec, row_spec, row_spec]
    args = [inp, g1, rev]
    if matmul:
        in_specs.append(pl.BlockSpec((2 * hh, 2 * hh), lambda i: (0, 0)))
        args.append(w_h_t)
    return pl.pallas_call(
        body,
        grid=(n // _BOND_BLK,),
        in_specs=in_specs,
        out_specs=row_spec,
        out_shape=jax.ShapeDtypeStruct((n, hh), jnp.int32),
    )(*args)


def _tc_readout(f_atoms, nei, wo1_t, wo2_t, bias):
    a, fdim = f_atoms.shape
    _, k, hh = nei.shape
    h = 2 * hh

    def body(fa_ref, n_ref, w1_ref, w2_ref, b_ref, o_ref):
        lo, hi = _unpack(n_ref[...])
        am_lo = jnp.sum(lo, axis=1) * (1.0 / k)
        am_hi = jnp.sum(hi, axis=1) * (1.0 / k)
        w2 = w2_ref[...]
        acc = jnp.dot(
            fa_ref[...].astype(_BF), w1_ref[...],
            preferred_element_type=jnp.float32,
        )
        acc += jnp.dot(
            am_lo.astype(_BF), w2[:hh], preferred_element_type=jnp.float32
        )
        acc += jnp.dot(
            am_hi.astype(_BF), w2[hh:], preferred_element_type=jnp.float32
        )
        o_ref[...] = jnp.maximum(acc + b_ref[...], 0.0)

    return pl.pallas_call(
        body,
        grid=(a // _ATOM_BLK,),
        in_specs=[
            pl.BlockSpec((_ATOM_BLK, fdim), lambda i: (i, 0)),
            pl.BlockSpec((_ATOM_BLK, k, hh), lambda i: (i, 0, 0)),
            pl.BlockSpec((fdim, h), lambda i: (0, 0)),
            pl.BlockSpec((h, h), lambda i: (0, 0)),
            pl.BlockSpec((1, h), lambda i: (0, 0)),
        ],
        out_specs=pl.BlockSpec((_ATOM_BLK, h), lambda i: (i, 0)),
        out_shape=jax.ShapeDtypeStruct((a, h), jnp.float32),
    )(f_atoms, nei, wo1_t, wo2_t, bias)


def kernel(f_atoms, f_bonds, a2b, b2a, b2revb, W_i, W_h, W_o_w, W_o_b):
    n_atoms, atom_fdim = f_atoms.shape
    max_nb = a2b.shape[1]
    h = W_i.shape[0]

    a2b_flat = a2b.reshape(-1).astype(jnp.int32)
    b2a = b2a.astype(jnp.int32)
    b2revb = b2revb.astype(jnp.int32)
    w_i_t = W_i.T.astype(_BF)
    w_h_t = W_h.T.astype(_BF)
    wo1_t = W_o_w[:, :atom_fdim].T.astype(_BF)
    wo2_t = W_o_w[:, atom_fdim:].T.astype(_BF)
    bias = W_o_b.reshape(1, h)

    hh = h // 2

    inp, msgh = _tc_init(f_bonds.astype(_BF), w_i_t, w_h_t)
    message = None
    for t in range(DEPTH - 1):
        nei = _sc_gather(msgh, a2b_flat)
        amh = _tc_segsum(nei.reshape(n_atoms, max_nb, hh))
        rev, g1 = _sc_gather_multi([(msgh, b2revb), (amh, b2a)])
        if t == DEPTH - 2:
            message = _tc_combine(inp, g1, rev, None)
        else:
            msgh = _tc_combine(inp, g1, rev, w_h_t)

    nei = _sc_gather(message, a2b_flat)
    out = _tc_readout(
        f_atoms, nei.reshape(n_atoms, max_nb, hh), wo1_t, wo2_t, bias
    )
    return out.reshape(N_MOLS, n_atoms // N_MOLS, h)


# speedup vs baseline: 1.0261x; 1.0261x over previous
import functools

import jax
import jax.numpy as jnp
from jax.experimental import pallas as pl
from jax.experimental.pallas import tpu as pltpu
from jax.experimental.pallas import tpu_sc as plsc

DEPTH = 3
N_MOLS = 100

_NC = 2
_NS = 16
_NW = _NC * _NS

_BF = jnp.bfloat16


def _pick_ring(per_w):
    for g in range(128, 0, -8):
        if per_w % g == 0:
            for nb in (5, 4, 3, 2, 1):
                if per_w % (g * nb) == 0:
                    return g, nb
    raise ValueError(per_w)


def _rt16(x):
    u = jax.lax.bitcast_convert_type(x, jnp.uint32)
    u = u + jnp.uint32(0x7FFF) + ((u >> 16) & jnp.uint32(1))
    return u >> 16


def _pack2(lo, hi):
    return jax.lax.bitcast_convert_type(
        _rt16(lo) | (_rt16(hi) << 16), jnp.int32
    )


def _unpack(p):
    u = jax.lax.bitcast_convert_type(p, jnp.uint32)
    lo = jax.lax.bitcast_convert_type(u << 16, jnp.float32)
    hi = jax.lax.bitcast_convert_type(
        u & jnp.uint32(0xFFFF0000), jnp.float32
    )
    return lo, hi


def _sc_gather_multi(jobs):
    n = jobs[0][1].shape[0]
    d = jobs[0][0].shape[1]
    per_w = n // _NW
    assert per_w * _NW == n and per_w % 8 == 0
    for t_, i_ in jobs:
        assert i_.shape[0] == n and t_.shape[1] == d
    gchunk, nbuf = _pick_ring(per_w)
    group = gchunk * nbuf
    n_groups = per_w // group
    mesh = plsc.VectorSubcoreMesh(core_axis_name="c", subcore_axis_name="s")
    njobs = len(jobs)

    @functools.partial(
        pl.kernel,
        mesh=mesh,
        out_type=[
            jax.ShapeDtypeStruct((n, d), jobs[0][0].dtype)
            for _ in range(njobs)
        ],
        scratch_types=[
            pltpu.VMEM((group,), jnp.int32),
            pltpu.VMEM((nbuf, gchunk, d), jobs[0][0].dtype),
            pltpu.SemaphoreType.DMA,
        ]
        + [pltpu.SemaphoreType.DMA] * nbuf,
    )
    def k(*refs):
        tables = refs[:njobs]
        idxs = refs[njobs:2 * njobs]
        outs = refs[2 * njobs:3 * njobs]
        idx_v, rows_v, gsem = refs[3 * njobs:3 * njobs + 3]
        wsems = refs[3 * njobs + 3:]
        wid = jax.lax.axis_index("s") * _NC + jax.lax.axis_index("c")
        base = wid * per_w

        def do_group(table_hbm, idx_hbm, out_hbm, g, wait_wb):
            gbase = pl.multiple_of(base + g * group, 8)
            if wait_wb:
                for b in range(nbuf):
                    off = pl.multiple_of(gbase + b * gchunk, 8)
                    pltpu.make_async_copy(
                        rows_v.at[b], out_hbm.at[pl.ds(off, gchunk)], wsems[b]
                    ).wait()
            pltpu.sync_copy(idx_hbm.at[pl.ds(gbase, group)], idx_v)
            handles = [
                pltpu.async_copy(
                    table_hbm.at[idx_v.at[pl.ds(b * gchunk, gchunk)]],
                    rows_v.at[b],
                    gsem,
                )
                for b in range(nbuf)
            ]
            for b, h in enumerate(handles):
                h.wait()
                off = pl.multiple_of(gbase + b * gchunk, 8)
                pltpu.async_copy(
                    rows_v.at[b], out_hbm.at[pl.ds(off, gchunk)], wsems[b]
                )

        for j in range(njobs):
            do_group(tables[j], idxs[j], outs[j], 0, wait_wb=(j > 0))

            @pl.loop(1, n_groups)
            def _(g, _j=j):
                do_group(tables[_j], idxs[_j], outs[_j], g, True)

        last = pl.multiple_of(base + (n_groups - 1) * group, 8)
        for b in range(nbuf):
            off = pl.multiple_of(last + b * gchunk, 8)
            pltpu.make_async_copy(
                rows_v.at[b], outs[-1].at[pl.ds(off, gchunk)], wsems[b]
            ).wait()

    outs = k(*[t for t, _ in jobs], *[i for _, i in jobs])
    if not isinstance(outs, (list, tuple)):
        outs = [outs]
    return list(outs)


def _sc_gather(table, idx):
    return _sc_gather_multi([(table, idx)])[0]


_BOND_BLK = 2000
_ATOM_BLK = 200


def _tc_init(f_bonds, w_i_t, w_h_t):
    n, fdim = f_bonds.shape
    h = w_i_t.shape[1]
    hh = h // 2

    def body(fb, wi, wh, inp_ref, msgh_ref):
        inp = jnp.dot(
            fb[...], wi[...], preferred_element_type=jnp.float32
        )
        inp_ref[...] = _pack2(inp[:, :hh], inp[:, hh:])
        msgh = jnp.dot(
            jnp.maximum(inp, 0.0).astype(_BF), wh[...],
            preferred_element_type=jnp.float32,
        )
        msgh_ref[...] = _pack2(msgh[:, :hh], msgh[:, hh:])

    return pl.pallas_call(
        body,
        grid=(n // _BOND_BLK,),
        in_specs=[
            pl.BlockSpec((_BOND_BLK, fdim), lambda i: (i, 0)),
            pl.BlockSpec((fdim, h), lambda i: (0, 0)),
            pl.BlockSpec((h, h), lambda i: (0, 0)),
        ],
        out_specs=[
            pl.BlockSpec((_BOND_BLK, hh), lambda i: (i, 0)),
            pl.BlockSpec((_BOND_BLK, hh), lambda i: (i, 0)),
        ],
        out_shape=[
            jax.ShapeDtypeStruct((n, hh), jnp.int32),
            jax.ShapeDtypeStruct((n, hh), jnp.int32),
        ],
    )(f_bonds, w_i_t, w_h_t)


def _tc_segsum(nei):
    a, k, hh = nei.shape

    def body(n_ref, o_ref):
        lo, hi = _unpack(n_ref[...])
        o_ref[...] = _pack2(jnp.sum(lo, axis=1), jnp.sum(hi, axis=1))

    return pl.pallas_call(
        body,
        grid=(a // _ATOM_BLK,),
        in_specs=[pl.BlockSpec((_ATOM_BLK, k, hh), lambda i: (i, 0, 0))],
        out_specs=pl.BlockSpec((_ATOM_BLK, hh), lambda i: (i, 0)),
        out_shape=jax.ShapeDtypeStruct((a, hh), jnp.int32),
    )(nei)


def _tc_combine(inp, g1, rev, w_h_t):
    n, hh = inp.shape
    matmul = w_h_t is not None

    def body(*refs):
        if matmul:
            inp_ref, g1_ref, rev_ref, wh_ref, o_ref = refs
        else:
            inp_ref, g1_ref, rev_ref, o_ref = refs
        i_lo, i_hi = _unpack(inp_ref[...])
        g_lo, g_hi = _unpack(g1_ref[...])
        r_lo, r_hi = _unpack(rev_ref[...])
        m_lo = jnp.maximum(i_lo + g_lo - r_lo, 0.0)
        m_hi = jnp.maximum(i_hi + g_hi - r_hi, 0.0)
        if matmul:
            wh = wh_ref[...]
            out = jnp.dot(
                m_lo.astype(_BF), wh[:hh], preferred_element_type=jnp.float32
            ) + jnp.dot(
                m_hi.astype(_BF), wh[hh:], preferred_element_type=jnp.float32
            )
            o_ref[...] = _pack2(out[:, :hh], out[:, hh:])
        else:
            o_ref[...] = _pack2(m_lo, m_hi)

    row_spec = pl.BlockSpec((_BOND_BLK, hh), lambda i: (i, 0))
    in_specs = [row_spec, row_spec, row_spec]
    args = [inp, g1, rev]
    if matmul:
        in_specs.append(pl.BlockSpec((2 * hh, 2 * hh), lambda i: (0, 0)))
        args.append(w_h_t)
    return pl.pallas_call(
        body,
        grid=(n // _BOND_BLK,),
        in_specs=in_specs,
        out_specs=row_spec,
        out_shape=jax.ShapeDtypeStruct((n, hh), jnp.int32),
    )(*args)


def _tc_readout(f_atoms, nei, wo1_t, wo2_t, bias):
    a, fdim = f_atoms.shape
    _, k, hh = nei.shape
    h = 2 * hh

    def body(fa_ref, n_ref, w1_ref, w2_ref, b_ref, o_ref):
        lo, hi = _unpack(n_ref[...])
        am_lo = jnp.sum(lo, axis=1) * (1.0 / k)
        am_hi = jnp.sum(hi, axis=1) * (1.0 / k)
        w2 = w2_ref[...]
        acc = jnp.dot(
            fa_ref[...].astype(_BF), w1_ref[...],
            preferred_element_type=jnp.float32,
        )
        acc += jnp.dot(
            am_lo.astype(_BF), w2[:hh], preferred_element_type=jnp.float32
        )
        acc += jnp.dot(
            am_hi.astype(_BF), w2[hh:], preferred_element_type=jnp.float32
        )
        o_ref[...] = jnp.maximum(acc + b_ref[...], 0.0)

    return pl.pallas_call(
        body,
        grid=(a // _ATOM_BLK,),
        in_specs=[
            pl.BlockSpec((_ATOM_BLK, fdim), lambda i: (i, 0)),
            pl.BlockSpec((_ATOM_BLK, k, hh), lambda i: (i, 0, 0)),
            pl.BlockSpec((fdim, h), lambda i: (0, 0)),
            pl.BlockSpec((h, h), lambda i: (0, 0)),
            pl.BlockSpec((1, h), lambda i: (0, 0)),
        ],
        out_specs=pl.BlockSpec((_ATOM_BLK, h), lambda i: (i, 0)),
        out_shape=jax.ShapeDtypeStruct((a, h), jnp.float32),
    )(f_atoms, nei, wo1_t, wo2_t, bias)


def kernel(f_atoms, f_bonds, a2b, b2a, b2revb, W_i, W_h, W_o_w, W_o_b):
    n_atoms, atom_fdim = f_atoms.shape
    max_nb = a2b.shape[1]
    h = W_i.shape[0]

    a2b_flat = a2b.reshape(-1).astype(jnp.int32)
    b2a = b2a.astype(jnp.int32)
    b2revb = b2revb.astype(jnp.int32)
    w_i_t = W_i.T.astype(_BF)
    w_h_t = W_h.T.astype(_BF)
    wo1_t = W_o_w[:, :atom_fdim].T.astype(_BF)
    wo2_t = W_o_w[:, atom_fdim:].T.astype(_BF)
    bias = W_o_b.reshape(1, h)

    hh = h // 2

    inp, msgh = _tc_init(f_bonds.astype(_BF), w_i_t, w_h_t)
    message = None
    for t in range(DEPTH - 1):
        nei = _sc_gather(msgh, a2b_flat)
        rev = _sc_gather(msgh, b2revb)
        amh = _tc_segsum(nei.reshape(n_atoms, max_nb, hh))
        g1 = _sc_gather(amh, b2a)
        if t == DEPTH - 2:
            message = _tc_combine(inp, g1, rev, None)
        else:
            msgh = _tc_combine(inp, g1, rev, w_h_t)

    nei = _sc_gather(message, a2b_flat)
    out = _tc_readout(
        f_atoms, nei.reshape(n_atoms, max_nb, hh), wo1_t, wo2_t, bias
    )
    return out.reshape(N_MOLS, n_atoms // N_MOLS, h)


# speedup vs baseline: 1.0947x; 1.0668x over previous
import functools

import jax
import jax.numpy as jnp
from jax.experimental import pallas as pl
from jax.experimental.pallas import tpu as pltpu
from jax.experimental.pallas import tpu_sc as plsc

DEPTH = 3
N_MOLS = 100

_NC = 2
_NS = 16
_NW = _NC * _NS

_BF = jnp.bfloat16


def _pick_ring(per_w):
    for g in range(128, 0, -8):
        if per_w % g == 0:
            for nb in (5, 4, 3, 2, 1):
                if per_w % (g * nb) == 0:
                    return g, nb
    raise ValueError(per_w)


def _rt16(x):
    u = jax.lax.bitcast_convert_type(x, jnp.uint32)
    u = u + jnp.uint32(0x7FFF) + ((u >> 16) & jnp.uint32(1))
    return u >> 16


def _pack2(lo, hi):
    return jax.lax.bitcast_convert_type(
        _rt16(lo) | (_rt16(hi) << 16), jnp.int32
    )


def _unpack(p):
    u = jax.lax.bitcast_convert_type(p, jnp.uint32)
    lo = jax.lax.bitcast_convert_type(u << 16, jnp.float32)
    hi = jax.lax.bitcast_convert_type(
        u & jnp.uint32(0xFFFF0000), jnp.float32
    )
    return lo, hi


def _sc_gather_multi(jobs):
    n = jobs[0][1].shape[0]
    d = jobs[0][0].shape[1]
    per_w = n // _NW
    assert per_w * _NW == n and per_w % 8 == 0
    for t_, i_ in jobs:
        assert i_.shape[0] == n and t_.shape[1] == d
    gchunk, nbuf = _pick_ring(per_w)
    group = gchunk * nbuf
    n_groups = per_w // group
    mesh = plsc.VectorSubcoreMesh(core_axis_name="c", subcore_axis_name="s")
    njobs = len(jobs)

    @functools.partial(
        pl.kernel,
        mesh=mesh,
        out_type=[
            jax.ShapeDtypeStruct((n, d), jobs[0][0].dtype)
            for _ in range(njobs)
        ],
        scratch_types=[
            pltpu.VMEM((group,), jnp.int32),
            pltpu.VMEM((nbuf, gchunk, d), jobs[0][0].dtype),
            pltpu.SemaphoreType.DMA,
        ]
        + [pltpu.SemaphoreType.DMA] * nbuf,
    )
    def k(*refs):
        tables = refs[:njobs]
        idxs = refs[njobs:2 * njobs]
        outs = refs[2 * njobs:3 * njobs]
        idx_v, rows_v, gsem = refs[3 * njobs:3 * njobs + 3]
        wsems = refs[3 * njobs + 3:]
        wid = jax.lax.axis_index("s") * _NC + jax.lax.axis_index("c")
        base = wid * per_w

        def do_group(table_hbm, idx_hbm, out_hbm, g, wait_wb):
            gbase = pl.multiple_of(base + g * group, 8)
            if wait_wb:
                for b in range(nbuf):
                    off = pl.multiple_of(gbase + b * gchunk, 8)
                    pltpu.make_async_copy(
                        rows_v.at[b], out_hbm.at[pl.ds(off, gchunk)], wsems[b]
                    ).wait()
            pltpu.sync_copy(idx_hbm.at[pl.ds(gbase, group)], idx_v)
            handles = [
                pltpu.async_copy(
                    table_hbm.at[idx_v.at[pl.ds(b * gchunk, gchunk)]],
                    rows_v.at[b],
                    gsem,
                )
                for b in range(nbuf)
            ]
            for b, h in enumerate(handles):
                h.wait()
                off = pl.multiple_of(gbase + b * gchunk, 8)
                pltpu.async_copy(
                    rows_v.at[b], out_hbm.at[pl.ds(off, gchunk)], wsems[b]
                )

        for j in range(njobs):
            do_group(tables[j], idxs[j], outs[j], 0, wait_wb=(j > 0))

            @pl.loop(1, n_groups)
            def _(g, _j=j):
                do_group(tables[_j], idxs[_j], outs[_j], g, True)

        last = pl.multiple_of(base + (n_groups - 1) * group, 8)
        for b in range(nbuf):
            off = pl.multiple_of(last + b * gchunk, 8)
            pltpu.make_async_copy(
                rows_v.at[b], outs[-1].at[pl.ds(off, gchunk)], wsems[b]
            ).wait()

    outs = k(*[t for t, _ in jobs], *[i for _, i in jobs])
    if not isinstance(outs, (list, tuple)):
        outs = [outs]
    return list(outs)


def _sc_gather(table, idx):
    return _sc_gather_multi([(table, idx)])[0]


_BOND_BLK = 4000
_ATOM_BLK = 400


def _tc_init(fb1, fb2, w_i1, w_i2, w_h_t):
    n, f1 = fb1.shape
    f2 = fb2.shape[1]
    h = w_h_t.shape[0]
    hh = h // 2

    def body(fb1_ref, fb2_ref, wi1, wi2, wh, inp_ref, msgh_ref):
        inp = jnp.dot(
            fb1_ref[...], wi1[...], preferred_element_type=jnp.float32
        )
        inp += jnp.dot(
            fb2_ref[...], wi2[...], preferred_element_type=jnp.float32
        )
        inp_ref[...] = _pack2(inp[:, :hh], inp[:, hh:])
        msgh = jnp.dot(
            jnp.maximum(inp, 0.0).astype(_BF), wh[...],
            preferred_element_type=jnp.float32,
        )
        msgh_ref[...] = _pack2(msgh[:, :hh], msgh[:, hh:])

    return pl.pallas_call(
        body,
        grid=(n // _BOND_BLK,),
        in_specs=[
            pl.BlockSpec((_BOND_BLK, f1), lambda i: (i, 0)),
            pl.BlockSpec((_BOND_BLK, f2), lambda i: (i, 0)),
            pl.BlockSpec((f1, h), lambda i: (0, 0)),
            pl.BlockSpec((f2, h), lambda i: (0, 0)),
            pl.BlockSpec((h, h), lambda i: (0, 0)),
        ],
        out_specs=[
            pl.BlockSpec((_BOND_BLK, hh), lambda i: (i, 0)),
            pl.BlockSpec((_BOND_BLK, hh), lambda i: (i, 0)),
        ],
        out_shape=[
            jax.ShapeDtypeStruct((n, hh), jnp.int32),
            jax.ShapeDtypeStruct((n, hh), jnp.int32),
        ],
    )(fb1, fb2, w_i1, w_i2, w_h_t)


def _tc_segsum(nei):
    a, k, hh = nei.shape

    def body(n_ref, o_ref):
        lo, hi = _unpack(n_ref[...])
        o_ref[...] = _pack2(jnp.sum(lo, axis=1), jnp.sum(hi, axis=1))

    return pl.pallas_call(
        body,
        grid=(a // _ATOM_BLK,),
        in_specs=[pl.BlockSpec((_ATOM_BLK, k, hh), lambda i: (i, 0, 0))],
        out_specs=pl.BlockSpec((_ATOM_BLK, hh), lambda i: (i, 0)),
        out_shape=jax.ShapeDtypeStruct((a, hh), jnp.int32),
    )(nei)


def _tc_combine(inp, g1, rev, w_h_t):
    n, hh = inp.shape
    matmul = w_h_t is not None

    def body(*refs):
        if matmul:
            inp_ref, g1_ref, rev_ref, wh_ref, o_ref = refs
        else:
            inp_ref, g1_ref, rev_ref, o_ref = refs
        i_lo, i_hi = _unpack(inp_ref[...])
        g_lo, g_hi = _unpack(g1_ref[...])
        r_lo, r_hi = _unpack(rev_ref[...])
        m_lo = jnp.maximum(i_lo + g_lo - r_lo, 0.0)
        m_hi = jnp.maximum(i_hi + g_hi - r_hi, 0.0)
        if matmul:
            wh = wh_ref[...]
            out = jnp.dot(
                m_lo.astype(_BF), wh[:hh], preferred_element_type=jnp.float32
            ) + jnp.dot(
                m_hi.astype(_BF), wh[hh:], preferred_element_type=jnp.float32
            )
            o_ref[...] = _pack2(out[:, :hh], out[:, hh:])
        else:
            o_ref[...] = _pack2(m_lo, m_hi)

    row_spec = pl.BlockSpec((_BOND_BLK, hh), lambda i: (i, 0))
    in_specs = [row_spec, row_spec, row_spec]
    args = [inp, g1, rev]
    if matmul:
        in_specs.append(pl.BlockSpec((2 * hh, 2 * hh), lambda i: (0, 0)))
        args.append(w_h_t)
    return pl.pallas_call(
        body,
        grid=(n // _BOND_BLK,),
        in_specs=in_specs,
        out_specs=row_spec,
        out_shape=jax.ShapeDtypeStruct((n, hh), jnp.int32),
    )(*args)


def _tc_readout(f_atoms, nei, wo1_t, wo2_t, bias):
    a, fdim = f_atoms.shape
    _, k, hh = nei.shape
    h = 2 * hh

    def body(fa_ref, n_ref, w1_ref, w2_ref, b_ref, o_ref):
        lo, hi = _unpack(n_ref[...])
        am_lo = jnp.sum(lo, axis=1) * (1.0 / k)
        am_hi = jnp.sum(hi, axis=1) * (1.0 / k)
        w2 = w2_ref[...]
        acc = jnp.dot(
            fa_ref[...].astype(_BF), w1_ref[...],
            preferred_element_type=jnp.float32,
        )
        acc += jnp.dot(
            am_lo.astype(_BF), w2[:hh], preferred_element_type=jnp.float32
        )
        acc += jnp.dot(
            am_hi.astype(_BF), w2[hh:], preferred_element_type=jnp.float32
        )
        o_ref[...] = jnp.maximum(acc + b_ref[...], 0.0)

    return pl.pallas_call(
        body,
        grid=(a // _ATOM_BLK,),
        in_specs=[
            pl.BlockSpec((_ATOM_BLK, fdim), lambda i: (i, 0)),
            pl.BlockSpec((_ATOM_BLK, k, hh), lambda i: (i, 0, 0)),
            pl.BlockSpec((fdim, h), lambda i: (0, 0)),
            pl.BlockSpec((h, h), lambda i: (0, 0)),
            pl.BlockSpec((1, h), lambda i: (0, 0)),
        ],
        out_specs=pl.BlockSpec((_ATOM_BLK, h), lambda i: (i, 0)),
        out_shape=jax.ShapeDtypeStruct((a, h), jnp.float32),
    )(f_atoms, nei, wo1_t, wo2_t, bias)


def kernel(f_atoms, f_bonds, a2b, b2a, b2revb, W_i, W_h, W_o_w, W_o_b):
    n_atoms, atom_fdim = f_atoms.shape
    max_nb = a2b.shape[1]
    h = W_i.shape[0]

    a2b_flat = a2b.reshape(-1).astype(jnp.int32)
    b2a = b2a.astype(jnp.int32)
    b2revb = b2revb.astype(jnp.int32)
    fb1 = f_bonds[:, :128].astype(_BF)
    fb2 = f_bonds[:, 128:].astype(_BF)
    w_i1 = W_i.T[:128].astype(_BF)
    w_i2 = W_i.T[128:].astype(_BF)
    w_h_t = W_h.T.astype(_BF)
    wo1_t = W_o_w[:, :atom_fdim].T.astype(_BF)
    wo2_t = W_o_w[:, atom_fdim:].T.astype(_BF)
    bias = W_o_b.reshape(1, h)

    hh = h // 2

    inp, msgh = _tc_init(fb1, fb2, w_i1, w_i2, w_h_t)
    message = None
    for t in range(DEPTH - 1):
        nei = _sc_gather(msgh, a2b_flat)
        rev = _sc_gather(msgh, b2revb)
        amh = _tc_segsum(nei.reshape(n_atoms, max_nb, hh))
        g1 = _sc_gather(amh, b2a)
        if t == DEPTH - 2:
            message = _tc_combine(inp, g1, rev, None)
        else:
            msgh = _tc_combine(inp, g1, rev, w_h_t)

    nei = _sc_gather(message, a2b_flat)
    out = _tc_readout(
        f_atoms, nei.reshape(n_atoms, max_nb, hh), wo1_t, wo2_t, bias
    )
    return out.reshape(N_MOLS, n_atoms // N_MOLS, h)
